# baseline (speedup 1.0000x reference)
import jax
import jax.numpy as jnp
from jax import lax
from jax.experimental import pallas as pl
from jax.experimental.pallas import tpu as pltpu
from jax.experimental.pallas import tpu_sc as plsc

N = 10000
E = 320000
D = 128
NC = 2
NS = 16
NW = NC * NS
EPW = E // NW
CHUNK = 64
EPW_P = 10048
CHUNKS = EPW_P // CHUNK
N_PAD = 10240
RPT = N_PAD // NS
ZR = 32
NBR = 4
NBI = 6
GA = 3
PF = GA + 2


def _sc_body(feat, src, dst, zrow_h, zdeg_h, aggp, degp,
             sidx, didx, rowsv, zrowv, degv, agg_sh,
             semg0, semg1, semg2, semg3,
             semi0, semi1, semi2, semi3, semi4, semi5,
             sems0, sems1, sems2, sems3):
    cid = lax.axis_index("c")
    sid = lax.axis_index("s")
    wid = cid * NS + sid
    base = wid * EPW_P
    row0 = sid * RPT
    semg = (semg0, semg1, semg2, semg3)
    semi = (semi0, semi1, semi2, semi3, semi4, semi5)
    sems = (sems0, sems1, sems2, sems3)

    pltpu.sync_copy(zrow_h, zrowv)
    pltpu.sync_copy(zdeg_h, degv)
    for z in range(RPT // ZR):
        pltpu.sync_copy(zrowv, agg_sh.at[pl.ds(row0 + z * ZR, ZR)])
    plsc.subcore_barrier()

    ones16 = jnp.full((16,), 1.0, jnp.float32)

    def load_idx(c, b):
        off = base + c * CHUNK
        pltpu.async_copy(src.at[pl.ds(off, CHUNK)], sidx.at[b], semi[b])
        pltpu.async_copy(dst.at[pl.ds(off, CHUNK)], didx.at[b], semi[b])

    def wait_idx(b):
        pltpu.make_async_copy(src.at[pl.ds(0, CHUNK)], sidx.at[b], semi[b]).wait()
        pltpu.make_async_copy(dst.at[pl.ds(0, CHUNK)], didx.at[b], semi[b]).wait()

    def start_gather(rb, ib):
        pltpu.async_copy(feat.at[didx.at[ib]], rowsv.at[rb], semg[rb])

    def wait_gather(rb):
        pltpu.make_async_copy(feat.at[pl.ds(0, CHUNK)], rowsv.at[rb],
                              semg[rb]).wait()

    def start_scatter(rb, ib):
        pltpu.async_copy(rowsv.at[rb], agg_sh.at[sidx.at[ib]], sems[rb],
                         add=True)

    def wait_scatter(rb):
        pltpu.make_async_copy(rowsv.at[rb], agg_sh.at[pl.ds(0, CHUNK)],
                              sems[rb]).wait()

    for k in range(PF):
        load_idx(k, k)
    for k in range(GA):
        wait_idx(k)
        start_gather(k, k)

    def body(c, carry):
        def step(rb, ib):
            grb, gib = (rb + GA) % NBR, (ib + GA) % NBI
            wait_gather(rb)

            @pl.when(c >= NBR - GA)
            def _():
                wait_scatter(grb)

            @pl.when(c < CHUNKS - GA)
            def _():
                wait_idx(gib)
                start_gather(grb, gib)

            start_scatter(rb, ib)
            for k in range(CHUNK // 16):
                idxv = sidx[ib, pl.ds(k * 16, 16)]
                plsc.addupdate_scatter(degv, [idxv], ones16)

            @pl.when(c < CHUNKS - PF)
            def _():
                load_idx(c + PF, (ib + PF) % NBI)

        r12 = lax.rem(c, 12)
        for m in range(12):
            @pl.when(r12 == m)
            def _(m=m):
                step(m % NBR, m % NBI)

        return carry

    lax.fori_loop(0, CHUNKS, body, 0)
    for k in range(CHUNKS - (NBR - GA), CHUNKS):
        wait_scatter(k % NBR)
    plsc.subcore_barrier()

    pltpu.sync_copy(agg_sh.at[pl.ds(row0, RPT)],
                    aggp.at[cid].at[pl.ds(row0, RPT)])
    pltpu.sync_copy(degv, degp.at[wid])


_sc_call = pl.kernel(
    _sc_body,
    out_type=[
        jax.ShapeDtypeStruct((NC, N_PAD, D), jnp.float32),
        jax.ShapeDtypeStruct((NW, N_PAD), jnp.float32),
    ],
    mesh=plsc.VectorSubcoreMesh(core_axis_name="c", subcore_axis_name="s"),
    compiler_params=pltpu.CompilerParams(needs_layout_passes=False),
    scratch_types=[
        pltpu.VMEM((NBI, CHUNK), jnp.int32),
        pltpu.VMEM((NBI, CHUNK), jnp.int32),
        pltpu.VMEM((NBR, CHUNK, D), jnp.float32),
        pltpu.VMEM((ZR, D), jnp.float32),
        pltpu.VMEM((N_PAD,), jnp.float32),
        pltpu.VMEM_SHARED((N_PAD, D), jnp.float32),
    ] + [pltpu.SemaphoreType.DMA] * (NBR + NBI + NBR),
)


def _tc_body(p_ref, dg_ref, x_ref, ww_ref, bw_ref, bias_ref, o_ref):
    agg = p_ref[0] + p_ref[1]
    deg = jnp.sum(dg_ref[...], axis=0)
    inv = jnp.where(deg > 0, 1.0 / deg, 0.0)
    h = agg * inv[:, None]
    out = lax.dot_general(h, ww_ref[...], (((1,), (1,)), ((), ())),
                          preferred_element_type=jnp.float32)
    out = out + lax.dot_general(x_ref[...], bw_ref[...], (((1,), (1,)), ((), ())),
                                preferred_element_type=jnp.float32)
    o_ref[...] = out + bias_ref[...]


BLK = 1024


def _tc_combine(aggp, degp, x, ww, bw, bias):
    grid = (N_PAD // BLK,)
    return pl.pallas_call(
        _tc_body,
        grid=grid,
        in_specs=[
            pl.BlockSpec((NC, BLK, D), lambda i: (0, i, 0)),
            pl.BlockSpec((NW, BLK), lambda i: (0, i)),
            pl.BlockSpec((BLK, D), lambda i: (i, 0)),
            pl.BlockSpec((D, D), lambda i: (0, 0)),
            pl.BlockSpec((D, D), lambda i: (0, 0)),
            pl.BlockSpec((1, D), lambda i: (0, 0)),
        ],
        out_specs=pl.BlockSpec((BLK, D), lambda i: (i, 0)),
        out_shape=jax.ShapeDtypeStruct((N_PAD, D), jnp.float32),
    )(aggp, degp, x, ww, bw, bias)


def kernel(features, edge_index, W_w, W_b, B_w, B_b):
    srcp = jnp.concatenate(
        [edge_index[0].reshape(NW, EPW),
         jnp.full((NW, EPW_P - EPW), N_PAD - 1, jnp.int32)], axis=1).reshape(-1)
    dstp = jnp.concatenate(
        [edge_index[1].reshape(NW, EPW),
         jnp.zeros((NW, EPW_P - EPW), jnp.int32)], axis=1).reshape(-1)
    zrow_h = jnp.zeros((ZR, D), jnp.float32)
    zdeg_h = jnp.zeros((N_PAD,), jnp.float32)
    aggp, degp = _sc_call(features, srcp, dstp, zrow_h, zdeg_h)
    bias = (W_b + B_b).reshape(1, D)
    return _tc_combine(aggp, degp, features, W_w, B_w, bias)[:N]


# speedup vs baseline: 1.4165x; 1.4165x over previous
import jax
import jax.numpy as jnp
from jax import lax
from jax.experimental import pallas as pl
from jax.experimental.pallas import tpu as pltpu
from jax.experimental.pallas import tpu_sc as plsc

N = 10000
E = 320000
D = 128
NC = 2
NS = 16
NW = NC * NS
EPW = E // NW
CHUNK = 80
CHUNKS = EPW // CHUNK
N_PAD = 10240
RPT = N_PAD // NS
ZR = 40


def _sc_body(feat, src, dst, zrow_h, zdeg_h, aggp, degp,
             sidx, didx, rowsv, zrowv, degv, agg_sh,
             semg0, semg1, semg2, semi0, semi1, semi2, semi3, semi4, semi5,
             sems0, sems1, sems2):
    cid = lax.axis_index("c")
    sid = lax.axis_index("s")
    wid = cid * NS + sid
    base = wid * EPW
    row0 = sid * RPT
    semg = (semg0, semg1, semg2)
    semi = (semi0, semi1, semi2, semi3, semi4, semi5)
    sems = (sems0, sems1, sems2)

    pltpu.sync_copy(zrow_h, zrowv)
    pltpu.sync_copy(zdeg_h, degv)
    for z in range(RPT // ZR):
        pltpu.sync_copy(zrowv, agg_sh.at[pl.ds(row0 + z * ZR, ZR)])
    plsc.subcore_barrier()

    ones16 = jnp.full((16,), 1.0, jnp.float32)

    def load_idx(c, b):
        off = base + c * CHUNK
        pltpu.async_copy(src.at[pl.ds(off, CHUNK)], sidx.at[b], semi[b])
        pltpu.async_copy(dst.at[pl.ds(off, CHUNK)], didx.at[b], semi[b])

    def wait_idx(b):
        pltpu.make_async_copy(src.at[pl.ds(0, CHUNK)], sidx.at[b], semi[b]).wait()
        pltpu.make_async_copy(dst.at[pl.ds(0, CHUNK)], didx.at[b], semi[b]).wait()

    def start_gather(rb, ib):
        pltpu.async_copy(feat.at[didx.at[ib]], rowsv.at[rb], semg[rb])

    def wait_gather(rb):
        pltpu.make_async_copy(feat.at[pl.ds(0, CHUNK)], rowsv.at[rb], semg[rb]).wait()

    def start_scatter(rb, ib):
        pltpu.async_copy(rowsv.at[rb], agg_sh.at[sidx.at[ib]], sems[rb],
                         add=True)

    def wait_scatter(rb):
        pltpu.make_async_copy(rowsv.at[rb], agg_sh.at[pl.ds(0, CHUNK)],
                              sems[rb]).wait()

    load_idx(0, 0)
    load_idx(1, 1)
    load_idx(2, 2)
    load_idx(3, 3)
    wait_idx(0)
    start_gather(0, 0)
    wait_idx(1)
    start_gather(1, 1)

    def body(c, carry):
        def step(rb, ib):
            g2rb, g2ib = (rb + 2) % 3, (ib + 2) % 6
            wait_gather(rb)

            @pl.when(c >= 1)
            def _():
                wait_scatter(g2rb)

            @pl.when(c < CHUNKS - 2)
            def _():
                wait_idx(g2ib)
                start_gather(g2rb, g2ib)

            start_scatter(rb, ib)
            for k in range(CHUNK // 16):
                idxv = sidx[ib, pl.ds(k * 16, 16)]
                plsc.addupdate_scatter(degv, [idxv], ones16)

            @pl.when(c < CHUNKS - 4)
            def _():
                load_idx(c + 4, (ib + 4) % 6)

        r6 = lax.rem(c, 6)
        for m in range(6):
            @pl.when(r6 == m)
            def _(m=m):
                step(m % 3, m % 6)

        return carry

    lax.fori_loop(0, CHUNKS, body, 0)
    wait_scatter((CHUNKS - 1) % 3)
    plsc.subcore_barrier()

    pltpu.sync_copy(agg_sh.at[pl.ds(row0, RPT)],
                    aggp.at[cid].at[pl.ds(row0, RPT)])
    pltpu.sync_copy(degv, degp.at[wid])


_sc_call = pl.kernel(
    _sc_body,
    out_type=[
        jax.ShapeDtypeStruct((NC, N_PAD, D), jnp.float32),
        jax.ShapeDtypeStruct((NW, N_PAD), jnp.float32),
    ],
    mesh=plsc.VectorSubcoreMesh(core_axis_name="c", subcore_axis_name="s"),
    compiler_params=pltpu.CompilerParams(needs_layout_passes=False),
    scratch_types=[
        pltpu.VMEM((6, CHUNK), jnp.int32),
        pltpu.VMEM((6, CHUNK), jnp.int32),
        pltpu.VMEM((3, CHUNK, D), jnp.float32),
        pltpu.VMEM((ZR, D), jnp.float32),
        pltpu.VMEM((N_PAD,), jnp.float32),
        pltpu.VMEM_SHARED((N_PAD, D), jnp.float32),
        pltpu.SemaphoreType.DMA,
        pltpu.SemaphoreType.DMA,
        pltpu.SemaphoreType.DMA,
        pltpu.SemaphoreType.DMA,
        pltpu.SemaphoreType.DMA,
        pltpu.SemaphoreType.DMA,
        pltpu.SemaphoreType.DMA,
        pltpu.SemaphoreType.DMA,
        pltpu.SemaphoreType.DMA,
        pltpu.SemaphoreType.DMA,
        pltpu.SemaphoreType.DMA,
        pltpu.SemaphoreType.DMA,
    ],
)


def _tc_body(p_ref, dg_ref, x_ref, ww_ref, bw_ref, bias_ref, o_ref):
    agg = p_ref[0] + p_ref[1]
    deg = jnp.sum(dg_ref[...], axis=0)
    inv = jnp.where(deg > 0, 1.0 / deg, 0.0)
    h = agg * inv[:, None]
    out = lax.dot_general(h, ww_ref[...], (((1,), (1,)), ((), ())),
                          preferred_element_type=jnp.float32)
    out = out + lax.dot_general(x_ref[...], bw_ref[...], (((1,), (1,)), ((), ())),
                                preferred_element_type=jnp.float32)
    o_ref[...] = out + bias_ref[...]


BLK = 1024


def _tc_combine(aggp, degp, x, ww, bw, bias):
    grid = (N_PAD // BLK,)
    return pl.pallas_call(
        _tc_body,
        grid=grid,
        in_specs=[
            pl.BlockSpec((NC, BLK, D), lambda i: (0, i, 0)),
            pl.BlockSpec((NW, BLK), lambda i: (0, i)),
            pl.BlockSpec((BLK, D), lambda i: (i, 0)),
            pl.BlockSpec((D, D), lambda i: (0, 0)),
            pl.BlockSpec((D, D), lambda i: (0, 0)),
            pl.BlockSpec((1, D), lambda i: (0, 0)),
        ],
        out_specs=pl.BlockSpec((BLK, D), lambda i: (i, 0)),
        out_shape=jax.ShapeDtypeStruct((N, D), jnp.float32),
    )(aggp, degp, x, ww, bw, bias)


def kernel(features, edge_index, W_w, W_b, B_w, B_b):
    src = edge_index[0]
    dst = edge_index[1]
    zrow_h = jnp.zeros((ZR, D), jnp.float32)
    zdeg_h = jnp.zeros((N_PAD,), jnp.float32)
    aggp, degp = _sc_call(features, src, dst, zrow_h, zdeg_h)
    bias = (W_b + B_b).reshape(1, D)
    return _tc_combine(aggp, degp, features, W_w, B_w, bias)


# speedup vs baseline: 1.4197x; 1.0023x over previous
import jax
import jax.numpy as jnp
from jax import lax
from jax.experimental import pallas as pl
from jax.experimental.pallas import tpu as pltpu
from jax.experimental.pallas import tpu_sc as plsc

N = 10000
E = 320000
D = 128
NC = 2
NS = 16
NW = NC * NS
EPW = E // NW
CHUNK = 80
CHUNKS = EPW // CHUNK
N_PAD = 10240
RPT = N_PAD // NS
ZR = 40


def _sc_body(feat, src, dst, zrow_h, zdeg_h, aggp, degp,
             sidx, didx, rowsv, zrowv, degv, agg_sh,
             semg0, semg1, semg2, semi0, semi1, semi2, semi3, semi4, semi5,
             sems0, sems1, sems2):
    cid = lax.axis_index("c")
    sid = lax.axis_index("s")
    wid = cid * NS + sid
    base = wid * EPW
    row0 = sid * RPT
    semg = (semg0, semg1, semg2)
    semi = (semi0, semi1, semi2, semi3, semi4, semi5)
    sems = (sems0, sems1, sems2)

    ones16 = jnp.full((16,), 1.0, jnp.float32)

    def load_idx(c, b):
        off = base + c * CHUNK
        pltpu.async_copy(src.at[pl.ds(off, CHUNK)], sidx.at[b], semi[b])
        pltpu.async_copy(dst.at[pl.ds(off, CHUNK)], didx.at[b], semi[b])

    def wait_idx(b):
        pltpu.make_async_copy(src.at[pl.ds(0, CHUNK)], sidx.at[b], semi[b]).wait()
        pltpu.make_async_copy(dst.at[pl.ds(0, CHUNK)], didx.at[b], semi[b]).wait()

    def start_gather(rb, ib):
        pltpu.async_copy(feat.at[didx.at[ib]], rowsv.at[rb], semg[rb])

    def wait_gather(rb):
        pltpu.make_async_copy(feat.at[pl.ds(0, CHUNK)], rowsv.at[rb], semg[rb]).wait()

    def start_scatter(rb, ib):
        pltpu.async_copy(rowsv.at[rb], agg_sh.at[sidx.at[ib]], sems[rb],
                         add=True)

    def wait_scatter(rb):
        pltpu.make_async_copy(rowsv.at[rb], agg_sh.at[pl.ds(0, CHUNK)],
                              sems[rb]).wait()

    load_idx(0, 0)
    load_idx(1, 1)
    load_idx(2, 2)
    load_idx(3, 3)
    wait_idx(0)
    start_gather(0, 0)
    wait_idx(1)
    start_gather(1, 1)

    pltpu.sync_copy(zrow_h, zrowv)
    pltpu.sync_copy(zdeg_h, degv)
    for z in range(RPT // ZR):
        pltpu.sync_copy(zrowv, agg_sh.at[pl.ds(row0 + z * ZR, ZR)])
    plsc.subcore_barrier()

    def body(c, carry):
        def step(rb, ib):
            g2rb, g2ib = (rb + 2) % 3, (ib + 2) % 6
            wait_gather(rb)

            @pl.when(c >= 1)
            def _():
                wait_scatter(g2rb)

            @pl.when(c < CHUNKS - 2)
            def _():
                wait_idx(g2ib)
                start_gather(g2rb, g2ib)

            start_scatter(rb, ib)
            for k in range(CHUNK // 16):
                idxv = sidx[ib, pl.ds(k * 16, 16)]
                plsc.addupdate_scatter(degv, [idxv], ones16)

            @pl.when(c < CHUNKS - 4)
            def _():
                load_idx(c + 4, (ib + 4) % 6)

        r6 = lax.rem(c, 6)
        for m in range(6):
            @pl.when(r6 == m)
            def _(m=m):
                step(m % 3, m % 6)

        return carry

    lax.fori_loop(0, CHUNKS, body, 0)
    wait_scatter((CHUNKS - 1) % 3)
    plsc.subcore_barrier()

    pltpu.sync_copy(agg_sh.at[pl.ds(row0, RPT)],
                    aggp.at[cid].at[pl.ds(row0, RPT)])
    pltpu.sync_copy(degv, degp.at[wid])


_sc_call = pl.kernel(
    _sc_body,
    out_type=[
        jax.ShapeDtypeStruct((NC, N_PAD, D), jnp.float32),
        jax.ShapeDtypeStruct((NW, N_PAD), jnp.float32),
    ],
    mesh=plsc.VectorSubcoreMesh(core_axis_name="c", subcore_axis_name="s"),
    compiler_params=pltpu.CompilerParams(needs_layout_passes=False),
    scratch_types=[
        pltpu.VMEM((6, CHUNK), jnp.int32),
        pltpu.VMEM((6, CHUNK), jnp.int32),
        pltpu.VMEM((3, CHUNK, D), jnp.float32),
        pltpu.VMEM((ZR, D), jnp.float32),
        pltpu.VMEM((N_PAD,), jnp.float32),
        pltpu.VMEM_SHARED((N_PAD, D), jnp.float32),
        pltpu.SemaphoreType.DMA,
        pltpu.SemaphoreType.DMA,
        pltpu.SemaphoreType.DMA,
        pltpu.SemaphoreType.DMA,
        pltpu.SemaphoreType.DMA,
        pltpu.SemaphoreType.DMA,
        pltpu.SemaphoreType.DMA,
        pltpu.SemaphoreType.DMA,
        pltpu.SemaphoreType.DMA,
        pltpu.SemaphoreType.DMA,
        pltpu.SemaphoreType.DMA,
        pltpu.SemaphoreType.DMA,
    ],
)


def _tc_body(p_ref, dg_ref, x_ref, ww_ref, bw_ref, bias_ref, o_ref):
    agg = p_ref[0] + p_ref[1]
    deg = jnp.sum(dg_ref[...], axis=0)
    inv = jnp.where(deg > 0, 1.0 / deg, 0.0)
    h = agg * inv[:, None]
    out = lax.dot_general(h, ww_ref[...], (((1,), (1,)), ((), ())),
                          preferred_element_type=jnp.float32)
    out = out + lax.dot_general(x_ref[...], bw_ref[...], (((1,), (1,)), ((), ())),
                                preferred_element_type=jnp.float32)
    o_ref[...] = out + bias_ref[...]


BLK = 1024


def _tc_combine(aggp, degp, x, ww, bw, bias):
    grid = (N_PAD // BLK,)
    return pl.pallas_call(
        _tc_body,
        grid=grid,
        in_specs=[
            pl.BlockSpec((NC, BLK, D), lambda i: (0, i, 0)),
            pl.BlockSpec((NW, BLK), lambda i: (0, i)),
            pl.BlockSpec((BLK, D), lambda i: (i, 0)),
            pl.BlockSpec((D, D), lambda i: (0, 0)),
            pl.BlockSpec((D, D), lambda i: (0, 0)),
            pl.BlockSpec((1, D), lambda i: (0, 0)),
        ],
        out_specs=pl.BlockSpec((BLK, D), lambda i: (i, 0)),
        out_shape=jax.ShapeDtypeStruct((N, D), jnp.float32),
    )(aggp, degp, x, ww, bw, bias)


def kernel(features, edge_index, W_w, W_b, B_w, B_b):
    src = edge_index[0]
    dst = edge_index[1]
    zrow_h = jnp.zeros((ZR, D), jnp.float32)
    zdeg_h = jnp.zeros((N_PAD,), jnp.float32)
    aggp, degp = _sc_call(features, src, dst, zrow_h, zdeg_h)
    bias = (W_b + B_b).reshape(1, D)
    return _tc_combine(aggp, degp, features, W_w, B_w, bias)
